# baseline (speedup 1.0000x reference)
import functools

import jax
import jax.numpy as jnp
from jax import lax
from jax.experimental import pallas as pl
from jax.experimental.pallas import tpu as pltpu
from jax.experimental.pallas import tpu_sc as plsc

_INFO = plsc.get_sparse_core_info()
_NC = _INFO.num_cores
_NS = _INFO.num_subcores
_NL = _INFO.num_lanes
_NW = _NC * _NS

_BATCH = 16384
_DIM = 64
_BPW = _BATCH // _NW
_CHUNK = 128
_NCHUNK = _BPW // _CHUNK


def _rsqrt(x):
    xi = plsc.bitcast(x, jnp.int32)
    y = plsc.bitcast(jnp.int32(0x5F3759DF) - (xi >> 1), jnp.float32)
    for _ in range(3):
        y = y * (1.5 - 0.5 * x * y * y)
    return y


def _body(bh, bt, br, ent, rel, out, idx_h, idx_t, idx_r,
          h_rows, t_rows, r_rows, out_v, sem_i, s0, s1, s2, s3):
    wid = lax.axis_index("s") * _NC + lax.axis_index("c")
    sems = (s0, s1, s2, s3)

    cbase = wid * _NCHUNK
    ci = [pltpu.async_copy(src.at[pl.ds(cbase, _NCHUNK)], dst, sem_i)
          for src, dst in ((bh, idx_h), (bt, idx_t), (br, idx_r))]
    for cp in ci:
        cp.wait()

    cps = []
    for c in range(_NCHUNK):
        rows = pl.ds(c * _CHUNK, _CHUNK)
        cps.append([
            pltpu.async_copy(ent.at[idx_h.at[c]], h_rows.at[rows], sems[c]),
            pltpu.async_copy(ent.at[idx_t.at[c]], t_rows.at[rows], sems[c]),
            pltpu.async_copy(rel.at[idx_r.at[c]], r_rows.at[rows], sems[c]),
        ])

    zf = jnp.zeros((_NL,), jnp.float32)
    zi = jnp.zeros((_NL,), jnp.int32)
    lane = lax.iota(jnp.int32, _NL)

    for c in range(_NCHUNK):
        for cp in cps[c]:
            cp.wait()

        def group(gi, _, c=c):
            row0 = c * _CHUNK + gi * _NL
            ridx = row0 + lane

            def pass_a(jb, carry):
                hs, rs, ts = carry
                for jo in range(8):
                    cj = zi + (jb * 8 + jo)
                    hj = plsc.load_gather(h_rows, [ridx, cj])
                    tj = plsc.load_gather(t_rows, [ridx, cj])
                    rj = plsc.load_gather(r_rows, [ridx, cj])
                    hs = hs + hj * hj
                    ts = ts + tj * tj
                    rs = rs + rj * rj
                return hs, rs, ts

            hs, rs, ts = lax.fori_loop(0, _DIM // 8, pass_a, (zf, zf, zf))
            ih = _rsqrt(jnp.maximum(hs, 1e-24))
            ir = _rsqrt(jnp.maximum(rs, 1e-24))
            it = _rsqrt(jnp.maximum(ts, 1e-24))

            def pass_b(jb, score):
                for jo in range(8):
                    cj = zi + (jb * 8 + jo)
                    hj = plsc.load_gather(h_rows, [ridx, cj])
                    tj = plsc.load_gather(t_rows, [ridx, cj])
                    rj = plsc.load_gather(r_rows, [ridx, cj])
                    score = score + jnp.abs(hj * ih + rj * ir - tj * it)
                return score

            score = lax.fori_loop(0, _DIM // 8, pass_b, zf)
            out_v[pl.ds(row0, _NL)] = score
            return 0

        lax.fori_loop(0, _CHUNK // _NL, group, 0)

    pltpu.sync_copy(out_v, out.at[pl.ds(wid * _BPW, _BPW)])


def kernel(batch_h, batch_t, batch_r, ent_emb, rel_emb):
    mesh = plsc.VectorSubcoreMesh(core_axis_name="c", subcore_axis_name="s")
    f = functools.partial(
        pl.kernel,
        mesh=mesh,
        compiler_params=pltpu.CompilerParams(
            needs_layout_passes=False, use_tc_tiling_on_sc=False),
        out_type=jax.ShapeDtypeStruct((_BATCH,), jnp.float32),
        scratch_types=[
            pltpu.VMEM((_NCHUNK, _CHUNK), jnp.int32),
            pltpu.VMEM((_NCHUNK, _CHUNK), jnp.int32),
            pltpu.VMEM((_NCHUNK, _CHUNK), jnp.int32),
            pltpu.VMEM((_BPW, _DIM), jnp.float32),
            pltpu.VMEM((_BPW, _DIM), jnp.float32),
            pltpu.VMEM((_BPW, _DIM), jnp.float32),
            pltpu.VMEM((_BPW,), jnp.float32),
            pltpu.SemaphoreType.DMA,
            pltpu.SemaphoreType.DMA,
            pltpu.SemaphoreType.DMA,
            pltpu.SemaphoreType.DMA,
            pltpu.SemaphoreType.DMA,
        ],
    )(_body)
    shape2 = (_NW * _NCHUNK, _CHUNK)
    return f(batch_h.reshape(shape2), batch_t.reshape(shape2),
             batch_r.reshape(shape2), ent_emb, rel_emb)


# speedup vs baseline: 1.1285x; 1.1285x over previous
import functools

import jax
import jax.numpy as jnp
from jax import lax
from jax.experimental import pallas as pl
from jax.experimental.pallas import tpu as pltpu
from jax.experimental.pallas import tpu_sc as plsc

_INFO = plsc.get_sparse_core_info()
_NC = _INFO.num_cores
_NS = _INFO.num_subcores
_NL = _INFO.num_lanes
_NW = _NC * _NS

_BATCH = 16384
_DIM = 64
_BPW = _BATCH // _NW
_CHUNK = 128
_NCHUNK = _BPW // _CHUNK


def _rsqrt(x):
    xi = plsc.bitcast(x, jnp.int32)
    y = plsc.bitcast(jnp.int32(0x5F3759DF) - (xi >> 1), jnp.float32)
    for _ in range(3):
        y = y * (1.5 - 0.5 * x * y * y)
    return y


def _body(bh, bt, br, ent, rel, out, idx_h, idx_t, idx_r,
          h_rows, t_rows, r_rows, out_v, sem_i, s0, s1, s2, s3):
    wid = lax.axis_index("s") * _NC + lax.axis_index("c")
    sems = (s0, s1, s2, s3)

    cbase = wid * _NCHUNK
    ci = [pltpu.async_copy(src.at[pl.ds(cbase, _NCHUNK)], dst, sem_i)
          for src, dst in ((bh, idx_h), (bt, idx_t), (br, idx_r))]
    for cp in ci:
        cp.wait()

    cps = []
    for c in range(_NCHUNK):
        rows = pl.ds(c * _CHUNK, _CHUNK)
        cps.append([
            pltpu.async_copy(ent.at[idx_h.at[c]], h_rows.at[rows], sems[c]),
            pltpu.async_copy(ent.at[idx_t.at[c]], t_rows.at[rows], sems[c]),
            pltpu.async_copy(rel.at[idx_r.at[c]], r_rows.at[rows], sems[c]),
        ])

    zf = jnp.zeros((_NL,), jnp.float32)
    zi = jnp.zeros((_NL,), jnp.int32)
    lane = lax.iota(jnp.int32, _NL)

    for c in range(_NCHUNK):
        for cp in cps[c]:
            cp.wait()

        def group(gi, _, c=c):
            row0 = c * _CHUNK + gi * _NL
            ridx = row0 + lane

            def pass_a(jb, carry):
                hs, rs, ts = carry
                for jo in range(8):
                    cj = (lane + (jb * 8 + jo)) & (_DIM - 1)
                    hj = plsc.load_gather(h_rows, [ridx, cj])
                    tj = plsc.load_gather(t_rows, [ridx, cj])
                    rj = plsc.load_gather(r_rows, [ridx, cj])
                    hs = hs + hj * hj
                    ts = ts + tj * tj
                    rs = rs + rj * rj
                return hs, rs, ts

            hs, rs, ts = lax.fori_loop(0, _DIM // 8, pass_a, (zf, zf, zf))
            ih = _rsqrt(jnp.maximum(hs, 1e-24))
            ir = _rsqrt(jnp.maximum(rs, 1e-24))
            it = _rsqrt(jnp.maximum(ts, 1e-24))

            def pass_b(jb, score):
                for jo in range(8):
                    cj = (lane + (jb * 8 + jo)) & (_DIM - 1)
                    hj = plsc.load_gather(h_rows, [ridx, cj])
                    tj = plsc.load_gather(t_rows, [ridx, cj])
                    rj = plsc.load_gather(r_rows, [ridx, cj])
                    score = score + jnp.abs(hj * ih + rj * ir - tj * it)
                return score

            score = lax.fori_loop(0, _DIM // 8, pass_b, zf)
            out_v[pl.ds(row0, _NL)] = score
            return 0

        lax.fori_loop(0, _CHUNK // _NL, group, 0)

    pltpu.sync_copy(out_v, out.at[pl.ds(wid * _BPW, _BPW)])


def kernel(batch_h, batch_t, batch_r, ent_emb, rel_emb):
    mesh = plsc.VectorSubcoreMesh(core_axis_name="c", subcore_axis_name="s")
    f = functools.partial(
        pl.kernel,
        mesh=mesh,
        compiler_params=pltpu.CompilerParams(
            needs_layout_passes=False, use_tc_tiling_on_sc=False),
        out_type=jax.ShapeDtypeStruct((_BATCH,), jnp.float32),
        scratch_types=[
            pltpu.VMEM((_NCHUNK, _CHUNK), jnp.int32),
            pltpu.VMEM((_NCHUNK, _CHUNK), jnp.int32),
            pltpu.VMEM((_NCHUNK, _CHUNK), jnp.int32),
            pltpu.VMEM((_BPW, _DIM), jnp.float32),
            pltpu.VMEM((_BPW, _DIM), jnp.float32),
            pltpu.VMEM((_BPW, _DIM), jnp.float32),
            pltpu.VMEM((_BPW,), jnp.float32),
            pltpu.SemaphoreType.DMA,
            pltpu.SemaphoreType.DMA,
            pltpu.SemaphoreType.DMA,
            pltpu.SemaphoreType.DMA,
            pltpu.SemaphoreType.DMA,
        ],
    )(_body)
    shape2 = (_NW * _NCHUNK, _CHUNK)
    return f(batch_h.reshape(shape2), batch_t.reshape(shape2),
             batch_r.reshape(shape2), ent_emb, rel_emb)


# speedup vs baseline: 1.2592x; 1.1159x over previous
import functools

import jax
import jax.numpy as jnp
from jax import lax
from jax.experimental import pallas as pl
from jax.experimental.pallas import tpu as pltpu
from jax.experimental.pallas import tpu_sc as plsc

_INFO = plsc.get_sparse_core_info()
_NC = _INFO.num_cores
_NS = _INFO.num_subcores
_NL = _INFO.num_lanes
_NW = _NC * _NS

_BATCH = 16384
_DIM = 64
_PDIM = 128
_BPW = _BATCH // _NW
_CHUNK = 128
_NCHUNK = _BPW // _CHUNK


def _rsqrt(x):
    xi = plsc.bitcast(x, jnp.int32)
    y = plsc.bitcast(jnp.int32(0x5F3759DF) - (xi >> 1), jnp.float32)
    for _ in range(3):
        y = y * (1.5 - 0.5 * x * y * y)
    return y


def _body(bh, bt, br, ent, rel, out, idx_h, idx_t, idx_r,
          hb, tb, rb, out_v, sem_i, s0, s1):
    wid = lax.axis_index("s") * _NC + lax.axis_index("c")
    base = wid * _BPW

    ci = [pltpu.async_copy(src.at[pl.ds(base, _BPW)], dst, sem_i)
          for src, dst in ((bh, idx_h), (bt, idx_t), (br, idx_r))]
    for cp in ci:
        cp.wait()

    sems = (s0, s1)
    lane = lax.iota(jnp.int32, _NL)
    zf = jnp.zeros((_NL,), jnp.float32)

    def gather(c, buf):
        rows = pl.ds(c * _CHUNK, _CHUNK)
        return [
            pltpu.async_copy(ent.at[idx_h.at[rows]], hb.at[buf], sems[buf]),
            pltpu.async_copy(ent.at[idx_t.at[rows]], tb.at[buf], sems[buf]),
            pltpu.async_copy(rel.at[idx_r.at[rows]], rb.at[buf], sems[buf]),
        ]

    pend = gather(0, 0)
    for c in range(_NCHUNK):
        for cp in pend:
            cp.wait()
        cur = c % 2
        if c + 1 < _NCHUNK:
            pend = gather(c + 1, 1 - cur)
        hc, tc, rc = hb.at[cur], tb.at[cur], rb.at[cur]

        def group(gi, _, hc=hc, tc=tc, rc=rc, c=c):
            ridx = gi * _NL + lane

            def pass_a(jb, carry):
                hs, rs, ts = carry
                for jo in range(8):
                    cj = (lane + (jb * 8 + jo)) & (_DIM - 1)
                    hj = plsc.load_gather(hc, [ridx, cj])
                    tj = plsc.load_gather(tc, [ridx, cj])
                    rj = plsc.load_gather(rc, [ridx, cj])
                    hs = hs + hj * hj
                    ts = ts + tj * tj
                    rs = rs + rj * rj
                return hs, rs, ts

            hs, rs, ts = lax.fori_loop(0, _DIM // 8, pass_a, (zf, zf, zf))
            ih = _rsqrt(jnp.maximum(hs, 1e-24))
            ir = _rsqrt(jnp.maximum(rs, 1e-24))
            it = _rsqrt(jnp.maximum(ts, 1e-24))

            def pass_b(jb, score):
                for jo in range(8):
                    cj = (lane + (jb * 8 + jo)) & (_DIM - 1)
                    hj = plsc.load_gather(hc, [ridx, cj])
                    tj = plsc.load_gather(tc, [ridx, cj])
                    rj = plsc.load_gather(rc, [ridx, cj])
                    score = score + jnp.abs(hj * ih + rj * ir - tj * it)
                return score

            score = lax.fori_loop(0, _DIM // 8, pass_b, zf)
            out_v[pl.ds(c * _CHUNK + gi * _NL, _NL)] = score
            return 0

        lax.fori_loop(0, _CHUNK // _NL, group, 0)

    pltpu.sync_copy(out_v, out.at[pl.ds(base, _BPW)])


def kernel(batch_h, batch_t, batch_r, ent_emb, rel_emb):
    mesh = plsc.VectorSubcoreMesh(core_axis_name="c", subcore_axis_name="s")
    f = functools.partial(
        pl.kernel,
        mesh=mesh,
        compiler_params=pltpu.CompilerParams(
            needs_layout_passes=False, use_tc_tiling_on_sc=True),
        out_type=jax.ShapeDtypeStruct((_BATCH,), jnp.float32),
        scratch_types=[
            pltpu.VMEM((_BPW,), jnp.int32),
            pltpu.VMEM((_BPW,), jnp.int32),
            pltpu.VMEM((_BPW,), jnp.int32),
            pltpu.VMEM((2, _CHUNK, _PDIM), jnp.float32),
            pltpu.VMEM((2, _CHUNK, _PDIM), jnp.float32),
            pltpu.VMEM((2, _CHUNK, _PDIM), jnp.float32),
            pltpu.VMEM((_BPW,), jnp.float32),
            pltpu.SemaphoreType.DMA,
            pltpu.SemaphoreType.DMA,
            pltpu.SemaphoreType.DMA,
        ],
    )(_body)
    pad = ((0, 0), (0, _PDIM - _DIM))
    return f(batch_h, batch_t, batch_r,
             jnp.pad(ent_emb, pad), jnp.pad(rel_emb, pad))
